# Initial kernel scaffold; baseline (speedup 1.0000x reference)
import jax
import jax.numpy as jnp
from jax.experimental import pallas as pl


def kernel(video_features, query_embedding, mu_W1, mu_b1, mu_W2, mu_b2, ln_w, ln_b, sg_W1, sg_b1, sg_W2, sg_b2, sel_W1, sel_b1, sel_W2, sel_b2, pos):
    raise NotImplementedError("write your pallas kernel here")


# speedup vs baseline: 5.7767x; 5.7767x over previous
import functools

import jax
import jax.numpy as jnp
from jax.experimental import pallas as pl

B, T, D, K_REP = 64, 1000, 256, 6
N_CAND = 12
BB = 8
MED_POS = (T - 1) // 2


def _gelu(x):
    return 0.5 * x * (1.0 + jax.lax.erf(x * (2.0 ** -0.5)))


def _prologue_kernel(q_ref, muW1T_ref, mub1_ref, muW2T_ref, mub2_ref,
                     lnw_ref, lnb_ref, sgW1T_ref, sgb1_ref, sgW2T_ref,
                     sgb2_ref, selW1bT_ref, selb1_ref,
                     muq_ref, sigma_ref, qpart_ref):
    q = q_ref[...]
    h = _gelu(jnp.dot(q, muW1T_ref[...], preferred_element_type=jnp.float32)
              + mub1_ref[...])
    x = jnp.dot(h, muW2T_ref[...], preferred_element_type=jnp.float32) + mub2_ref[...]
    m = jnp.mean(x, axis=-1, keepdims=True)
    v = jnp.mean((x - m) ** 2, axis=-1, keepdims=True)
    muq_ref[...] = (x - m) / jnp.sqrt(v + 1e-5) * lnw_ref[...] + lnb_ref[...]
    h2 = _gelu(jnp.dot(q, sgW1T_ref[...], preferred_element_type=jnp.float32)
               + sgb1_ref[...])
    s = jnp.dot(h2, sgW2T_ref[...], preferred_element_type=jnp.float32) + sgb2_ref[...]
    sigma_ref[...] = jax.nn.sigmoid(s) * 2.0 + 0.1
    qpart_ref[...] = (jnp.dot(q, selW1bT_ref[...], preferred_element_type=jnp.float32)
                      + selb1_ref[...])


def _main_kernel(video_ref, pos_ref, muq_ref, sigma_ref, qpart_ref,
                 selW1aT_ref, selw2_ref, selb2_ref,
                 feats_ref, dist_ref, idx_ref, rep_ref):
    f32 = jnp.float32
    feats = video_ref[...] + pos_ref[...][None, :, :]
    feats_ref[...] = feats

    muq = muq_ref[...]
    inv_s = 1.0 / (sigma_ref[...] + 1e-8)
    c = feats - muq[:, None, :]
    dist = jnp.sum(c * c * inv_s[:, None, :], axis=-1)
    dist = jnp.clip(dist, 1e-8, None)
    dist_ref[...] = dist

    flat = feats.reshape(BB * T, D)
    h = jnp.dot(flat, selW1aT_ref[...], preferred_element_type=f32)
    h = _gelu(h.reshape(BB, T, D) + qpart_ref[...][:, None, :])
    rel = jnp.sum(h * selw2_ref[...][None], axis=-1) + selb2_ref[0, 0]

    lo = jnp.min(dist, axis=-1, keepdims=True) - 1.0
    hi = jnp.max(dist, axis=-1, keepdims=True)

    def bisect(_, carry):
        lo, hi = carry
        mid = 0.5 * (lo + hi)
        cnt = jnp.sum((dist <= mid).astype(f32), axis=-1, keepdims=True)
        ge = cnt >= (MED_POS + 1)
        return jnp.where(ge, lo, mid), jnp.where(ge, mid, hi)

    lo, hi = jax.lax.fori_loop(0, 44, bisect, (lo, hi))
    target = jnp.min(jnp.where(dist > lo, dist, 1e30), axis=-1, keepdims=True)

    scores = -jnp.abs(dist - target) + 0.5 * rel

    t_iota = jax.lax.broadcasted_iota(jnp.int32, (BB, T), 1).astype(f32)
    lane = jax.lax.broadcasted_iota(jnp.int32, (BB, 128), 1).astype(f32)
    cand = jnp.full((BB, 128), 1e9, f32)
    s = scores
    for k in range(N_CAND):
        m = jnp.max(s, axis=-1, keepdims=True)
        best = jnp.min(jnp.where(s == m, t_iota, 1e9), axis=-1, keepdims=True)
        cand = jnp.where(lane == k, best, cand)
        s = jnp.where(t_iota == best, -1e30, s)

    less = (cand[:, :, None] < cand[:, None, :]).astype(f32)
    rank = jnp.sum(less, axis=1)
    onehot = (rank[:, :, None] == lane[:, None, :]).astype(f32)
    csort = jnp.sum(cand[:, :, None] * onehot, axis=1)

    c0 = csort[:, 0:1]
    out = jnp.where(lane == 0, c0, jnp.zeros_like(csort))
    mind = jnp.abs(csort - c0)
    avail = (lane < N_CAND) & (lane > 0)
    sel_vals = [c0]
    for step in range(1, K_REP):
        score = jnp.where(avail, mind, -1.0)
        m = jnp.max(score, axis=-1, keepdims=True)
        bl = jnp.min(jnp.where(score == m, lane, 1e9), axis=-1, keepdims=True)
        val = jnp.sum(jnp.where(lane == bl, csort, 0.0), axis=-1, keepdims=True)
        out = jnp.where(lane == step, val, out)
        avail = avail & (lane != bl)
        mind = jnp.minimum(mind, jnp.abs(csort - val))
        sel_vals.append(val)
    idx_ref[...] = out.astype(jnp.int32)

    for k in range(K_REP):
        oh = (t_iota == sel_vals[k]).astype(f32)
        row = jax.lax.dot_general(oh, feats, (((1,), (1,)), ((0,), (0,))),
                                  preferred_element_type=f32)
        rep_ref[:, k, :] = row


def kernel(video_features, query_embedding, mu_W1, mu_b1, mu_W2, mu_b2,
           ln_w, ln_b, sg_W1, sg_b1, sg_W2, sg_b2,
           sel_W1, sel_b1, sel_W2, sel_b2, pos):
    f32 = jnp.float32
    r2 = lambda v: v.reshape(1, -1)

    muq, sigma, qpart = pl.pallas_call(
        _prologue_kernel,
        out_shape=[jax.ShapeDtypeStruct((B, D), f32)] * 3,
    )(query_embedding, mu_W1.T, r2(mu_b1), mu_W2.T, r2(mu_b2),
      r2(ln_w), r2(ln_b), sg_W1.T, r2(sg_b1), sg_W2.T, r2(sg_b2),
      sel_W1[:, D:].T, r2(sel_b1))

    grid = (B // BB,)
    bspec = lambda shp: pl.BlockSpec(shp, lambda i: (i,) + (0,) * (len(shp) - 1))
    rep_full = lambda shp: pl.BlockSpec(shp, lambda i: (0,) * len(shp))

    feats, dist, idx_pad, rep = pl.pallas_call(
        _main_kernel,
        grid=grid,
        in_specs=[
            bspec((BB, T, D)),
            rep_full((T, D)),
            bspec((BB, D)),
            bspec((BB, D)),
            bspec((BB, D)),
            rep_full((D, D)),
            rep_full((1, D)),
            rep_full((1, 1)),
        ],
        out_specs=[
            bspec((BB, T, D)),
            bspec((BB, T)),
            bspec((BB, 128)),
            bspec((BB, K_REP, D)),
        ],
        out_shape=[
            jax.ShapeDtypeStruct((B, T, D), f32),
            jax.ShapeDtypeStruct((B, T), f32),
            jax.ShapeDtypeStruct((B, 128), jnp.int32),
            jax.ShapeDtypeStruct((B, K_REP, D), f32),
        ],
    )(video_features, pos[:T], muq, sigma, qpart,
      sel_W1[:, :D].T, sel_W2.reshape(1, D), sel_b2.reshape(1, 1))

    idx = idx_pad[:, :K_REP]
    return rep, idx, dist, muq, sigma, feats
